# Initial kernel scaffold; baseline (speedup 1.0000x reference)
import jax
import jax.numpy as jnp
from jax.experimental import pallas as pl


def kernel(x, pos_emb):
    raise NotImplementedError("write your pallas kernel here")


# speedup vs baseline: 1.4831x; 1.4831x over previous
import jax
import jax.numpy as jnp
from jax.experimental import pallas as pl


BLK = 512


def _add_kernel(x_ref, pos_ref, out_ref):
    out_ref[0] = x_ref[0] + pos_ref[...]


def kernel(x, pos_emb):
    b, n, d = x.shape
    num_s = n // BLK
    grid = (num_s, b)
    return pl.pallas_call(
        _add_kernel,
        grid=grid,
        in_specs=[
            pl.BlockSpec((1, BLK, d), lambda s, bb: (bb, s, 0)),
            pl.BlockSpec((BLK, d), lambda s, bb: (s, 0)),
        ],
        out_specs=pl.BlockSpec((1, BLK, d), lambda s, bb: (bb, s, 0)),
        out_shape=jax.ShapeDtypeStruct((b, n, d), x.dtype),
    )(x, pos_emb[:n])


# speedup vs baseline: 1.7377x; 1.1716x over previous
import jax
import jax.numpy as jnp
from jax.experimental import pallas as pl


BLK = 2048


def _add_kernel(x_ref, pos_ref, out_ref):
    out_ref[0] = x_ref[0] + pos_ref[...]


def kernel(x, pos_emb):
    b, n, d = x.shape
    num_s = n // BLK
    grid = (num_s, b)
    return pl.pallas_call(
        _add_kernel,
        grid=grid,
        in_specs=[
            pl.BlockSpec((1, BLK, d), lambda s, bb: (bb, s, 0)),
            pl.BlockSpec((BLK, d), lambda s, bb: (s, 0)),
        ],
        out_specs=pl.BlockSpec((1, BLK, d), lambda s, bb: (bb, s, 0)),
        out_shape=jax.ShapeDtypeStruct((b, n, d), x.dtype),
    )(x, pos_emb[:n])
